# Initial kernel scaffold; baseline (speedup 1.0000x reference)
import jax
import jax.numpy as jnp
from jax.experimental import pallas as pl


def kernel(x, batch_idx, W1, b1, g1, be1, W2, b2, g2, be2, W3, b3):
    raise NotImplementedError("write your pallas kernel here")


# speedup vs baseline: 4.2793x; 4.2793x over previous
import functools

import jax
import jax.numpy as jnp
from jax import lax
from jax.experimental import pallas as pl
from jax.experimental.pallas import tpu as pltpu
from jax.experimental.pallas import tpu_sc as plsc

N, D, B = 50000, 512, 64
L = 16
NC, NS = 2, 16
NW = NC * NS
CW = D // NW
CHUNK = 1000
NCHUNK = N // CHUNK


def _iload(ref, i):
    return ref[pl.ds(i, L)][0]


def _pool_body(x_hbm, idx_hbm, out_hbm, idx_v, buf, accm, accs, bnd, sem0, sem1):
    c = lax.axis_index("c")
    s_ = lax.axis_index("s")
    w = s_ * NC + c
    col0 = w * CW

    pltpu.sync_copy(idx_hbm, idx_v.at[pl.ds(0, N)])

    def init(i, carry):
        accm[i, :] = jnp.full((L,), -jnp.inf, jnp.float32)
        accs[i, :] = jnp.zeros((L,), jnp.float32)
        return carry

    lax.fori_loop(0, B, init, 0)

    bnd[0] = 0
    bnd[B] = N

    def bsearch(s, carry):
        def step(_, lohi):
            lo, hi = lohi
            mid = (lo + hi) // 2
            v = _iload(idx_v, mid)
            lo2 = jnp.where(v < s, mid + 1, lo)
            hi2 = jnp.where(v < s, hi, mid)
            return lo2, hi2

        lo, _ = lax.fori_loop(0, 16, step, (0, N))
        bnd[s] = lo
        return carry

    lax.fori_loop(1, B, bsearch, 0)

    sems = (sem0, sem1)

    def start(ci, slot):
        pltpu.async_copy(
            x_hbm.at[pl.ds(ci * CHUNK, CHUNK), pl.ds(col0, CW)],
            buf.at[slot], sems[slot])

    def wait(slot):
        pltpu.make_async_copy(
            x_hbm.at[pl.ds(0, CHUNK), pl.ds(col0, CW)],
            buf.at[slot], sems[slot]).wait()

    def process(ci, slot):
        r0 = ci * CHUNK
        s_lo = _iload(idx_v, r0)
        s_hi = _iload(idx_v, r0 + CHUNK - 1)

        def seg(s, carry):
            lo = jnp.maximum(bnd[s], r0)
            hi = jnp.minimum(bnd[s + 1], r0 + CHUNK)

            @pl.when(hi > lo)
            def _():
                mv = accm[s, :]
                sv = accs[s, :]

                def row(r, ms):
                    v = buf[slot, r - r0, :]
                    return jnp.maximum(ms[0], v), ms[1] + v

                mv2, sv2 = lax.fori_loop(lo, hi, row, (mv, sv))
                accm[s, :] = mv2
                accs[s, :] = sv2

            return carry

        lax.fori_loop(s_lo, s_hi + 1, seg, 0)

    start(0, 0)

    def outer(oi, carry):
        c0 = oi * 2
        start(c0 + 1, 1)
        wait(0)
        process(c0, 0)

        @pl.when(c0 + 2 < NCHUNK)
        def _():
            start(c0 + 2, 0)

        wait(1)
        process(c0 + 1, 1)
        return carry

    lax.fori_loop(0, NCHUNK // 2, outer, 0)

    def flush(s, carry):
        cnt = bnd[s + 1] - bnd[s]
        cntf = jnp.maximum(cnt, 1).astype(jnp.float32)
        accs[s, :] = accs[s, :] / jnp.full((L,), cntf, jnp.float32)
        return carry

    lax.fori_loop(0, B, flush, 0)

    pltpu.sync_copy(accm, out_hbm.at[pl.ds(0, B), pl.ds(col0, CW)])
    pltpu.sync_copy(accs, out_hbm.at[pl.ds(0, B), pl.ds(D + col0, CW)])


@jax.jit
def _pool(x, batch_idx):
    return pl.kernel(
        _pool_body,
        out_type=jax.ShapeDtypeStruct((B, 2 * D), jnp.float32),
        mesh=plsc.VectorSubcoreMesh(core_axis_name="c", subcore_axis_name="s",
                                    num_cores=NC, num_subcores=NS),
        scratch_types=[
            pltpu.VMEM((N + L,), jnp.int32),
            pltpu.VMEM((2, CHUNK, CW), jnp.float32),
            pltpu.VMEM((B, CW), jnp.float32),
            pltpu.VMEM((B, CW), jnp.float32),
            pltpu.SMEM((B + 1,), jnp.int32),
            pltpu.SemaphoreType.DMA,
            pltpu.SemaphoreType.DMA,
        ],
        compiler_params=pltpu.CompilerParams(use_tc_tiling_on_sc=False),
    )(x, batch_idx)


def _bn_lrelu(h, g, be):
    m = jnp.mean(h, axis=0, keepdims=True)
    v = jnp.mean(jnp.square(h - m), axis=0, keepdims=True)
    h = (h - m) * jax.lax.rsqrt(v + 1e-5) * g + be
    return jnp.where(h >= 0, h, 0.01 * h)


def _mlp_body(xx, W1, b1, g1, be1, W2, b2, g2, be2, W3, b3, out):
    h = jnp.dot(xx[:], W1[:], preferred_element_type=jnp.float32) + b1[:]
    h = _bn_lrelu(h, g1[:], be1[:])
    h = jnp.dot(h, W2[:], preferred_element_type=jnp.float32) + b2[:]
    h = _bn_lrelu(h, g2[:], be2[:])
    out[:] = jnp.dot(h, W3[:], preferred_element_type=jnp.float32) + b3[:]


@jax.jit
def _mlp(xx, W1, b1, g1, be1, W2, b2, g2, be2, W3, b3):
    OUT = W3.shape[1]
    return pl.pallas_call(
        _mlp_body,
        out_shape=jax.ShapeDtypeStruct((B, OUT), jnp.float32),
    )(xx, W1, b1.reshape(1, -1), g1.reshape(1, -1), be1.reshape(1, -1),
      W2, b2.reshape(1, -1), g2.reshape(1, -1), be2.reshape(1, -1),
      W3, b3.reshape(1, -1))


def kernel(x, batch_idx, W1, b1, g1, be1, W2, b2, g2, be2, W3, b3):
    xx = _pool(x, batch_idx)
    return _mlp(xx, W1, b1, g1, be1, W2, b2, g2, be2, W3, b3)


# speedup vs baseline: 7.9281x; 1.8527x over previous
import functools

import jax
import jax.numpy as jnp
from jax import lax
from jax.experimental import pallas as pl
from jax.experimental.pallas import tpu as pltpu
from jax.experimental.pallas import tpu_sc as plsc

N, D, B = 50000, 512, 64
L = 16
NC, NS = 2, 16
NW = NC * NS
CW = D // NW
CHUNK = 1000
NCHUNK = N // CHUNK


def _iload(ref, i):
    return ref[pl.ds(i, L)][0]


def _pool_body(x_hbm, idx_hbm, out_hbm, idx_v, buf, accm, accs, bnd, sem0, sem1):
    c = lax.axis_index("c")
    s_ = lax.axis_index("s")
    w = s_ * NC + c
    col0 = w * CW

    pltpu.sync_copy(idx_hbm, idx_v.at[pl.ds(0, N)])

    def init(i, carry):
        accm[i, :] = jnp.full((L,), -jnp.inf, jnp.float32)
        accs[i, :] = jnp.zeros((L,), jnp.float32)
        return carry

    lax.fori_loop(0, B, init, 0)

    bnd[0] = 0
    bnd[B] = N

    def bsearch(s, carry):
        def step(_, lohi):
            lo, hi = lohi
            mid = (lo + hi) // 2
            v = _iload(idx_v, mid)
            lo2 = jnp.where(v < s, mid + 1, lo)
            hi2 = jnp.where(v < s, hi, mid)
            return lo2, hi2

        lo, _ = lax.fori_loop(0, 16, step, (0, N))
        bnd[s] = lo
        return carry

    lax.fori_loop(1, B, bsearch, 0)

    sems = (sem0, sem1)

    def start(ci, slot):
        pltpu.async_copy(
            x_hbm.at[pl.ds(ci * CHUNK, CHUNK), pl.ds(col0, CW)],
            buf.at[slot], sems[slot])

    def wait(slot):
        pltpu.make_async_copy(
            x_hbm.at[pl.ds(0, CHUNK), pl.ds(col0, CW)],
            buf.at[slot], sems[slot]).wait()

    def process(ci, slot):
        r0 = ci * CHUNK
        s_lo = _iload(idx_v, r0)
        s_hi = _iload(idx_v, r0 + CHUNK - 1)

        def seg(s, carry):
            lo = jnp.maximum(bnd[s], r0)
            hi = jnp.minimum(bnd[s + 1], r0 + CHUNK)

            @pl.when(hi > lo)
            def _():
                mv = accm[s, :]
                sv = accs[s, :]
                base = lo - r0
                n8 = (hi - lo) // 8

                def row8(k, ms):
                    r = base + k * 8
                    v0 = buf[slot, r, :]
                    v1 = buf[slot, r + 1, :]
                    v2 = buf[slot, r + 2, :]
                    v3 = buf[slot, r + 3, :]
                    v4 = buf[slot, r + 4, :]
                    v5 = buf[slot, r + 5, :]
                    v6 = buf[slot, r + 6, :]
                    v7 = buf[slot, r + 7, :]
                    m = jnp.maximum(
                        jnp.maximum(jnp.maximum(v0, v1), jnp.maximum(v2, v3)),
                        jnp.maximum(jnp.maximum(v4, v5), jnp.maximum(v6, v7)))
                    t = ((v0 + v1) + (v2 + v3)) + ((v4 + v5) + (v6 + v7))
                    return jnp.maximum(ms[0], m), ms[1] + t

                mv2, sv2 = lax.fori_loop(0, n8, row8, (mv, sv))

                def row(r, ms):
                    v = buf[slot, r - r0, :]
                    return jnp.maximum(ms[0], v), ms[1] + v

                mv2, sv2 = lax.fori_loop(lo + n8 * 8, hi, row, (mv2, sv2))
                accm[s, :] = mv2
                accs[s, :] = sv2

            return carry

        lax.fori_loop(s_lo, s_hi + 1, seg, 0)

    start(0, 0)

    def outer(oi, carry):
        c0 = oi * 2
        start(c0 + 1, 1)
        wait(0)
        process(c0, 0)

        @pl.when(c0 + 2 < NCHUNK)
        def _():
            start(c0 + 2, 0)

        wait(1)
        process(c0 + 1, 1)
        return carry

    lax.fori_loop(0, NCHUNK // 2, outer, 0)

    def flush(s, carry):
        cnt = bnd[s + 1] - bnd[s]
        cntf = jnp.maximum(cnt, 1).astype(jnp.float32)
        accs[s, :] = accs[s, :] / jnp.full((L,), cntf, jnp.float32)
        return carry

    lax.fori_loop(0, B, flush, 0)

    pltpu.sync_copy(accm, out_hbm.at[pl.ds(0, B), pl.ds(col0, CW)])
    pltpu.sync_copy(accs, out_hbm.at[pl.ds(0, B), pl.ds(D + col0, CW)])


@jax.jit
def _pool(x, batch_idx):
    return pl.kernel(
        _pool_body,
        out_type=jax.ShapeDtypeStruct((B, 2 * D), jnp.float32),
        mesh=plsc.VectorSubcoreMesh(core_axis_name="c", subcore_axis_name="s",
                                    num_cores=NC, num_subcores=NS),
        scratch_types=[
            pltpu.VMEM((N + L,), jnp.int32),
            pltpu.VMEM((2, CHUNK, CW), jnp.float32),
            pltpu.VMEM((B, CW), jnp.float32),
            pltpu.VMEM((B, CW), jnp.float32),
            pltpu.SMEM((B + 1,), jnp.int32),
            pltpu.SemaphoreType.DMA,
            pltpu.SemaphoreType.DMA,
        ],
        compiler_params=pltpu.CompilerParams(use_tc_tiling_on_sc=False),
    )(x, batch_idx)


def _bn_lrelu(h, g, be):
    m = jnp.mean(h, axis=0, keepdims=True)
    v = jnp.mean(jnp.square(h - m), axis=0, keepdims=True)
    h = (h - m) * jax.lax.rsqrt(v + 1e-5) * g + be
    return jnp.where(h >= 0, h, 0.01 * h)


def _mlp_body(xx, W1, b1, g1, be1, W2, b2, g2, be2, W3, b3, out):
    h = jnp.dot(xx[:], W1[:], preferred_element_type=jnp.float32) + b1[:]
    h = _bn_lrelu(h, g1[:], be1[:])
    h = jnp.dot(h, W2[:], preferred_element_type=jnp.float32) + b2[:]
    h = _bn_lrelu(h, g2[:], be2[:])
    out[:] = jnp.dot(h, W3[:], preferred_element_type=jnp.float32) + b3[:]


@jax.jit
def _mlp(xx, W1, b1, g1, be1, W2, b2, g2, be2, W3, b3):
    OUT = W3.shape[1]
    return pl.pallas_call(
        _mlp_body,
        out_shape=jax.ShapeDtypeStruct((B, OUT), jnp.float32),
    )(xx, W1, b1.reshape(1, -1), g1.reshape(1, -1), be1.reshape(1, -1),
      W2, b2.reshape(1, -1), g2.reshape(1, -1), be2.reshape(1, -1),
      W3, b3.reshape(1, -1))


def kernel(x, batch_idx, W1, b1, g1, be1, W2, b2, g2, be2, W3, b3):
    xx = _pool(x, batch_idx)
    return _mlp(xx, W1, b1, g1, be1, W2, b2, g2, be2, W3, b3)


# speedup vs baseline: 14.8582x; 1.8741x over previous
import functools

import jax
import jax.numpy as jnp
from jax import lax
from jax.experimental import pallas as pl
from jax.experimental.pallas import tpu as pltpu
from jax.experimental.pallas import tpu_sc as plsc

N, D, B = 50000, 512, 64
L = 16
NC, NS = 2, 16
CBW = 128
KC = CBW // L
NG = 8
CHUNK = 208
NFULL = 30
ROWS_BIG = 6256
ROWS_SMALL = 6248
TAIL_BIG = ROWS_BIG - NFULL * CHUNK
TAIL_SMALL = ROWS_SMALL - NFULL * CHUNK
SEG_BLK = B // NG


def _iload(ref, i):
    return ref[pl.ds(i, L)][0]


def _pool_body(x_hbm, idx_hbm, out_hbm,
               idx_v, buf, accm, accs, cntb,
               tm, ts, tc, cm, cs, cc,
               bnd, sem0, sem1, pmax, psum, pcnt):
    c = lax.axis_index("c")
    s_ = lax.axis_index("s")
    cb_local = s_ // NG
    g = s_ % NG
    col0 = (c * 2 + cb_local) * CBW
    r_start = g * ROWS_SMALL + 8 * jnp.minimum(g, 2)
    r_len = jnp.where(g < 2, ROWS_BIG, ROWS_SMALL)

    @pl.when(g < 2)
    def _():
        pltpu.sync_copy(idx_hbm.at[pl.ds(r_start, ROWS_BIG)],
                        idx_v.at[pl.ds(0, ROWS_BIG)])

    @pl.when(g >= 2)
    def _():
        pltpu.sync_copy(idx_hbm.at[pl.ds(r_start, ROWS_SMALL)],
                        idx_v.at[pl.ds(0, ROWS_SMALL)])

    def init(i, carry):
        for k in range(KC):
            accm[i, pl.ds(k * L, L)] = jnp.full((L,), -jnp.inf, jnp.float32)
            accs[i, pl.ds(k * L, L)] = jnp.zeros((L,), jnp.float32)
        return carry

    lax.fori_loop(0, B, init, 0)

    bnd[0] = 0
    bnd[B] = r_len

    def bsearch(s, carry):
        def step(_, lohi):
            lo, hi = lohi
            mid = (lo + hi) // 2
            v = _iload(idx_v, mid)
            lo2 = jnp.where(v < s, mid + 1, lo)
            hi2 = jnp.where(v < s, hi, mid)
            return lo2, hi2

        lo, _ = lax.fori_loop(0, 13, step, (0, r_len))
        bnd[s] = lo
        return carry

    lax.fori_loop(1, B, bsearch, 0)

    def counts(s, carry):
        cnt = (bnd[s + 1] - bnd[s]).astype(jnp.float32)
        cntb[s // SEG_BLK, pl.ds((s % SEG_BLK) * L, L)] = jnp.full((L,), cnt, jnp.float32)
        return carry

    lax.fori_loop(0, B, counts, 0)

    sems = (sem0, sem1)

    def start(ci, slot):
        pltpu.async_copy(
            x_hbm.at[pl.ds(r_start + ci * CHUNK, CHUNK), pl.ds(col0, CBW)],
            buf.at[slot, pl.ds(0, CHUNK)], sems[slot])

    def wait(slot):
        pltpu.make_async_copy(
            x_hbm.at[pl.ds(0, CHUNK), pl.ds(col0, CBW)],
            buf.at[slot, pl.ds(0, CHUNK)], sems[slot]).wait()

    def process(lr0, nrows, slot):
        s_lo = _iload(idx_v, lr0)
        s_hi = _iload(idx_v, lr0 + nrows - 1)

        def seg(s, carry):
            lo = jnp.maximum(bnd[s], lr0)
            hi = jnp.minimum(bnd[s + 1], lr0 + nrows)

            @pl.when(hi > lo)
            def _():
                acc = ([accm[s, pl.ds(k * L, L)] for k in range(KC)]
                       + [accs[s, pl.ds(k * L, L)] for k in range(KC)])
                base = lo - lr0
                n2 = (hi - lo) // 2

                def row2(i, acc):
                    r = base + i * 2
                    out = []
                    for k in range(KC):
                        v0 = buf[slot, r, pl.ds(k * L, L)]
                        v1 = buf[slot, r + 1, pl.ds(k * L, L)]
                        out.append(jnp.maximum(acc[k], jnp.maximum(v0, v1)))
                        out.append(acc[KC + k] + (v0 + v1))
                    return tuple(out[0::2]) + tuple(out[1::2])

                acc = lax.fori_loop(0, n2, row2, tuple(acc))

                def row1(r, acc):
                    out = []
                    for k in range(KC):
                        v = buf[slot, r - lr0, pl.ds(k * L, L)]
                        out.append(jnp.maximum(acc[k], v))
                        out.append(acc[KC + k] + v)
                    return tuple(out[0::2]) + tuple(out[1::2])

                acc = lax.fori_loop(lo + n2 * 2, hi, row1, acc)
                for k in range(KC):
                    accm[s, pl.ds(k * L, L)] = acc[k]
                    accs[s, pl.ds(k * L, L)] = acc[KC + k]

            return carry

        lax.fori_loop(s_lo, s_hi + 1, seg, 0)

    start(0, 0)

    def outer(oi, carry):
        c0 = oi * 2
        start(c0 + 1, 1)
        wait(0)
        process(c0 * CHUNK, CHUNK, 0)

        @pl.when(c0 + 2 < NFULL)
        def _():
            start(c0 + 2, 0)

        wait(1)
        process((c0 + 1) * CHUNK, CHUNK, 1)
        return carry

    lax.fori_loop(0, NFULL // 2, outer, 0)

    @pl.when(g < 2)
    def _():
        pltpu.sync_copy(
            x_hbm.at[pl.ds(r_start + NFULL * CHUNK, TAIL_BIG), pl.ds(col0, CBW)],
            buf.at[0, pl.ds(0, TAIL_BIG)])
        process(NFULL * CHUNK, TAIL_BIG, 0)

    @pl.when(g >= 2)
    def _():
        pltpu.sync_copy(
            x_hbm.at[pl.ds(r_start + NFULL * CHUNK, TAIL_SMALL), pl.ds(col0, CBW)],
            buf.at[0, pl.ds(0, TAIL_SMALL)])
        process(NFULL * CHUNK, TAIL_SMALL, 0)

    pltpu.sync_copy(accm, pmax.at[s_])
    pltpu.sync_copy(accs, psum.at[s_])
    pltpu.sync_copy(cntb, pcnt.at[s_])
    plsc.subcore_barrier()

    j = s_ % NG
    seg0 = j * SEG_BLK
    ocol = (c * 2 + cb_local) * CBW

    for i in range(NG):
        p = cb_local * NG + i
        pltpu.sync_copy(pmax.at[p, pl.ds(seg0, SEG_BLK)], tm.at[i])
        pltpu.sync_copy(psum.at[p, pl.ds(seg0, SEG_BLK)], ts.at[i])
        pltpu.sync_copy(pcnt.at[p, j], tc.at[i])

    for i in range(NG):
        for k in range(KC):
            if i == 0:
                cc[pl.ds(k * L, L)] = tc[i, pl.ds(k * L, L)]
            else:
                cc[pl.ds(k * L, L)] = cc[pl.ds(k * L, L)] + tc[i, pl.ds(k * L, L)]
        for r in range(SEG_BLK):
            if i == 0:
                for k in range(KC):
                    cm[r, pl.ds(k * L, L)] = tm[i, r, pl.ds(k * L, L)]
                    cs[r, pl.ds(k * L, L)] = ts[i, r, pl.ds(k * L, L)]
            else:
                for k in range(KC):
                    cm[r, pl.ds(k * L, L)] = jnp.maximum(
                        cm[r, pl.ds(k * L, L)], tm[i, r, pl.ds(k * L, L)])
                    cs[r, pl.ds(k * L, L)] = (
                        cs[r, pl.ds(k * L, L)] + ts[i, r, pl.ds(k * L, L)])

    for r in range(SEG_BLK):
        inv = 1.0 / jnp.maximum(cc[pl.ds(r * L, L)], 1.0)
        for k in range(KC):
            cs[r, pl.ds(k * L, L)] = cs[r, pl.ds(k * L, L)] * inv

    pltpu.sync_copy(cm, out_hbm.at[pl.ds(seg0, SEG_BLK), pl.ds(ocol, CBW)])
    pltpu.sync_copy(cs, out_hbm.at[pl.ds(seg0, SEG_BLK), pl.ds(D + ocol, CBW)])


@jax.jit
def _pool(x, batch_idx):
    return pl.kernel(
        _pool_body,
        out_type=jax.ShapeDtypeStruct((B, 2 * D), jnp.float32),
        mesh=plsc.VectorSubcoreMesh(core_axis_name="c", subcore_axis_name="s",
                                    num_cores=NC, num_subcores=NS),
        scratch_types=[
            pltpu.VMEM((ROWS_BIG + L,), jnp.int32),
            pltpu.VMEM((2, CHUNK, CBW), jnp.float32),
            pltpu.VMEM((B, CBW), jnp.float32),
            pltpu.VMEM((B, CBW), jnp.float32),
            pltpu.VMEM((SEG_BLK, CBW), jnp.float32),
            pltpu.VMEM((NG, SEG_BLK, CBW), jnp.float32),
            pltpu.VMEM((NG, SEG_BLK, CBW), jnp.float32),
            pltpu.VMEM((NG, CBW), jnp.float32),
            pltpu.VMEM((SEG_BLK, CBW), jnp.float32),
            pltpu.VMEM((SEG_BLK, CBW), jnp.float32),
            pltpu.VMEM((CBW,), jnp.float32),
            pltpu.SMEM((B + 1,), jnp.int32),
            pltpu.SemaphoreType.DMA,
            pltpu.SemaphoreType.DMA,
            pltpu.VMEM_SHARED((NS, B, CBW), jnp.float32),
            pltpu.VMEM_SHARED((NS, B, CBW), jnp.float32),
            pltpu.VMEM_SHARED((NS, SEG_BLK, CBW), jnp.float32),
        ],
    )(x, batch_idx)


def _bn_lrelu(h, g, be):
    m = jnp.mean(h, axis=0, keepdims=True)
    v = jnp.mean(jnp.square(h - m), axis=0, keepdims=True)
    h = (h - m) * jax.lax.rsqrt(v + 1e-5) * g + be
    return jnp.where(h >= 0, h, 0.01 * h)


def _mlp_body(xx, W1, b1, g1, be1, W2, b2, g2, be2, W3, b3, out):
    h = jnp.dot(xx[:], W1[:], preferred_element_type=jnp.float32) + b1[:]
    h = _bn_lrelu(h, g1[:], be1[:])
    h = jnp.dot(h, W2[:], preferred_element_type=jnp.float32) + b2[:]
    h = _bn_lrelu(h, g2[:], be2[:])
    out[:] = jnp.dot(h, W3[:], preferred_element_type=jnp.float32) + b3[:]


@jax.jit
def _mlp(xx, W1, b1, g1, be1, W2, b2, g2, be2, W3, b3):
    OUT = W3.shape[1]
    return pl.pallas_call(
        _mlp_body,
        out_shape=jax.ShapeDtypeStruct((B, OUT), jnp.float32),
    )(xx, W1, b1.reshape(1, -1), g1.reshape(1, -1), be1.reshape(1, -1),
      W2, b2.reshape(1, -1), g2.reshape(1, -1), be2.reshape(1, -1),
      W3, b3.reshape(1, -1))


def kernel(x, batch_idx, W1, b1, g1, be1, W2, b2, g2, be2, W3, b3):
    xx = _pool(x, batch_idx)
    return _mlp(xx, W1, b1, g1, be1, W2, b2, g2, be2, W3, b3)


# speedup vs baseline: 15.2594x; 1.0270x over previous
import functools

import jax
import jax.numpy as jnp
from jax import lax
from jax.experimental import pallas as pl
from jax.experimental.pallas import tpu as pltpu
from jax.experimental.pallas import tpu_sc as plsc

N, D, B = 50000, 512, 64
L = 16
NC, NS = 2, 16
CBW = 128
KC = CBW // L
NG = 8
CHUNK = 208
NFULL = 30
ROWS_BIG = 6256
ROWS_SMALL = 6248
TAIL_BIG = ROWS_BIG - NFULL * CHUNK
TAIL_SMALL = ROWS_SMALL - NFULL * CHUNK
SEG_BLK = B // NG


def _iload(ref, i):
    return ref[pl.ds(i, L)][0]


def _pool_body(x_hbm, idx_hbm, out_hbm,
               idx_v, buf, accm, accs, cntb,
               tm, ts, tc, cm, cs, cc,
               bnd, sem0, sem1, pmax, psum, pcnt):
    c = lax.axis_index("c")
    s_ = lax.axis_index("s")
    cb_local = s_ // NG
    g = s_ % NG
    col0 = (c * 2 + cb_local) * CBW
    r_start = g * ROWS_SMALL + 8 * jnp.minimum(g, 2)
    r_len = jnp.where(g < 2, ROWS_BIG, ROWS_SMALL)

    @pl.when(g < 2)
    def _():
        pltpu.sync_copy(idx_hbm.at[pl.ds(r_start, ROWS_BIG)],
                        idx_v.at[pl.ds(0, ROWS_BIG)])

    @pl.when(g >= 2)
    def _():
        pltpu.sync_copy(idx_hbm.at[pl.ds(r_start, ROWS_SMALL)],
                        idx_v.at[pl.ds(0, ROWS_SMALL)])

    def init(i, carry):
        for k in range(KC):
            accm[i, pl.ds(k * L, L)] = jnp.full((L,), -jnp.inf, jnp.float32)
            accs[i, pl.ds(k * L, L)] = jnp.zeros((L,), jnp.float32)
        return carry

    lax.fori_loop(0, B, init, 0)

    bnd[0] = 0
    bnd[B] = r_len

    def bsearch(s, carry):
        def step(_, lohi):
            lo, hi = lohi
            mid = (lo + hi) // 2
            v = _iload(idx_v, mid)
            lo2 = jnp.where(v < s, mid + 1, lo)
            hi2 = jnp.where(v < s, hi, mid)
            return lo2, hi2

        lo, _ = lax.fori_loop(0, 13, step, (0, r_len))
        bnd[s] = lo
        return carry

    lax.fori_loop(1, B, bsearch, 0)

    def counts(s, carry):
        cnt = (bnd[s + 1] - bnd[s]).astype(jnp.float32)
        cntb[s // SEG_BLK, pl.ds((s % SEG_BLK) * L, L)] = jnp.full((L,), cnt, jnp.float32)
        return carry

    lax.fori_loop(0, B, counts, 0)

    sems = (sem0, sem1)

    def start(ci, slot):
        pltpu.async_copy(
            x_hbm.at[pl.ds(r_start + ci * CHUNK, CHUNK), pl.ds(col0, CBW)],
            buf.at[slot, pl.ds(0, CHUNK)], sems[slot])

    def wait(slot):
        pltpu.make_async_copy(
            x_hbm.at[pl.ds(0, CHUNK), pl.ds(col0, CBW)],
            buf.at[slot, pl.ds(0, CHUNK)], sems[slot]).wait()

    def process(lr0, nrows, slot):
        s_lo = _iload(idx_v, lr0)
        s_hi = _iload(idx_v, lr0 + nrows - 1)

        def seg(s, carry):
            lo = jnp.maximum(bnd[s], lr0)
            hi = jnp.minimum(bnd[s + 1], lr0 + nrows)

            @pl.when(hi > lo)
            def _():
                acc = ([accm[s, pl.ds(k * L, L)] for k in range(KC)]
                       + [accs[s, pl.ds(k * L, L)] for k in range(KC)])
                base = lo - lr0
                n2 = (hi - lo) // 2

                def row2(i, acc):
                    r = base + i * 2
                    out = []
                    for k in range(KC):
                        v0 = buf[slot, r, pl.ds(k * L, L)]
                        v1 = buf[slot, r + 1, pl.ds(k * L, L)]
                        out.append(jnp.maximum(acc[k], jnp.maximum(v0, v1)))
                        out.append(acc[KC + k] + (v0 + v1))
                    return tuple(out[0::2]) + tuple(out[1::2])

                acc = lax.fori_loop(0, n2, row2, tuple(acc))

                def row1(r, acc):
                    out = []
                    for k in range(KC):
                        v = buf[slot, r - lr0, pl.ds(k * L, L)]
                        out.append(jnp.maximum(acc[k], v))
                        out.append(acc[KC + k] + v)
                    return tuple(out[0::2]) + tuple(out[1::2])

                acc = lax.fori_loop(lo + n2 * 2, hi, row1, acc)
                for k in range(KC):
                    accm[s, pl.ds(k * L, L)] = acc[k]
                    accs[s, pl.ds(k * L, L)] = acc[KC + k]

            return carry

        lax.fori_loop(s_lo, s_hi + 1, seg, 0)

    start(0, 0)

    def outer(oi, carry):
        c0 = oi * 2
        start(c0 + 1, 1)
        wait(0)
        process(c0 * CHUNK, CHUNK, 0)

        @pl.when(c0 + 2 < NFULL)
        def _():
            start(c0 + 2, 0)

        wait(1)
        process((c0 + 1) * CHUNK, CHUNK, 1)
        return carry

    lax.fori_loop(0, NFULL // 2, outer, 0)

    @pl.when(g < 2)
    def _():
        pltpu.sync_copy(
            x_hbm.at[pl.ds(r_start + NFULL * CHUNK, TAIL_BIG), pl.ds(col0, CBW)],
            buf.at[0, pl.ds(0, TAIL_BIG)])
        process(NFULL * CHUNK, TAIL_BIG, 0)

    @pl.when(g >= 2)
    def _():
        pltpu.sync_copy(
            x_hbm.at[pl.ds(r_start + NFULL * CHUNK, TAIL_SMALL), pl.ds(col0, CBW)],
            buf.at[0, pl.ds(0, TAIL_SMALL)])
        process(NFULL * CHUNK, TAIL_SMALL, 0)

    pltpu.sync_copy(accm, pmax.at[s_])
    pltpu.sync_copy(accs, psum.at[s_])
    pltpu.sync_copy(cntb, pcnt.at[s_])
    plsc.subcore_barrier()

    j = s_ % NG
    seg0 = j * SEG_BLK
    ocol = (c * 2 + cb_local) * CBW

    descs = []
    for i in range(NG):
        p = cb_local * NG + i
        descs.append(pltpu.async_copy(pmax.at[p, pl.ds(seg0, SEG_BLK)], tm.at[i], sem0))
        descs.append(pltpu.async_copy(psum.at[p, pl.ds(seg0, SEG_BLK)], ts.at[i], sem0))
        descs.append(pltpu.async_copy(pcnt.at[p, j], tc.at[i], sem1))
    for d in descs:
        d.wait()

    for i in range(NG):
        for k in range(KC):
            if i == 0:
                cc[pl.ds(k * L, L)] = tc[i, pl.ds(k * L, L)]
            else:
                cc[pl.ds(k * L, L)] = cc[pl.ds(k * L, L)] + tc[i, pl.ds(k * L, L)]
        for r in range(SEG_BLK):
            if i == 0:
                for k in range(KC):
                    cm[r, pl.ds(k * L, L)] = tm[i, r, pl.ds(k * L, L)]
                    cs[r, pl.ds(k * L, L)] = ts[i, r, pl.ds(k * L, L)]
            else:
                for k in range(KC):
                    cm[r, pl.ds(k * L, L)] = jnp.maximum(
                        cm[r, pl.ds(k * L, L)], tm[i, r, pl.ds(k * L, L)])
                    cs[r, pl.ds(k * L, L)] = (
                        cs[r, pl.ds(k * L, L)] + ts[i, r, pl.ds(k * L, L)])

    for r in range(SEG_BLK):
        inv = 1.0 / jnp.maximum(cc[pl.ds(r * L, L)], 1.0)
        for k in range(KC):
            cs[r, pl.ds(k * L, L)] = cs[r, pl.ds(k * L, L)] * inv

    pltpu.sync_copy(cm, out_hbm.at[pl.ds(seg0, SEG_BLK), pl.ds(ocol, CBW)])
    pltpu.sync_copy(cs, out_hbm.at[pl.ds(seg0, SEG_BLK), pl.ds(D + ocol, CBW)])


def _pool(x, batch_idx):
    return pl.kernel(
        _pool_body,
        out_type=jax.ShapeDtypeStruct((B, 2 * D), jnp.float32),
        mesh=plsc.VectorSubcoreMesh(core_axis_name="c", subcore_axis_name="s",
                                    num_cores=NC, num_subcores=NS),
        scratch_types=[
            pltpu.VMEM((ROWS_BIG + L,), jnp.int32),
            pltpu.VMEM((2, CHUNK, CBW), jnp.float32),
            pltpu.VMEM((B, CBW), jnp.float32),
            pltpu.VMEM((B, CBW), jnp.float32),
            pltpu.VMEM((SEG_BLK, CBW), jnp.float32),
            pltpu.VMEM((NG, SEG_BLK, CBW), jnp.float32),
            pltpu.VMEM((NG, SEG_BLK, CBW), jnp.float32),
            pltpu.VMEM((NG, CBW), jnp.float32),
            pltpu.VMEM((SEG_BLK, CBW), jnp.float32),
            pltpu.VMEM((SEG_BLK, CBW), jnp.float32),
            pltpu.VMEM((CBW,), jnp.float32),
            pltpu.SMEM((B + 1,), jnp.int32),
            pltpu.SemaphoreType.DMA,
            pltpu.SemaphoreType.DMA,
            pltpu.VMEM_SHARED((NS, B, CBW), jnp.float32),
            pltpu.VMEM_SHARED((NS, B, CBW), jnp.float32),
            pltpu.VMEM_SHARED((NS, SEG_BLK, CBW), jnp.float32),
        ],
    )(x, batch_idx)


def _bn_lrelu(h, g, be):
    m = jnp.mean(h, axis=0, keepdims=True)
    v = jnp.mean(jnp.square(h - m), axis=0, keepdims=True)
    h = (h - m) * jax.lax.rsqrt(v + 1e-5) * g + be
    return jnp.where(h >= 0, h, 0.01 * h)


def _mlp_body(xx, W1, b1, g1, be1, W2, b2, g2, be2, W3, b3, out):
    h = jnp.dot(xx[:], W1[:], preferred_element_type=jnp.float32) + b1[:]
    h = _bn_lrelu(h, g1[:], be1[:])
    h = jnp.dot(h, W2[:], preferred_element_type=jnp.float32) + b2[:]
    h = _bn_lrelu(h, g2[:], be2[:])
    out[:] = jnp.dot(h, W3[:], preferred_element_type=jnp.float32) + b3[:]


def _mlp(xx, W1, b1, g1, be1, W2, b2, g2, be2, W3, b3):
    OUT = W3.shape[1]
    return pl.pallas_call(
        _mlp_body,
        out_shape=jax.ShapeDtypeStruct((B, OUT), jnp.float32),
    )(xx, W1, b1.reshape(1, -1), g1.reshape(1, -1), be1.reshape(1, -1),
      W2, b2.reshape(1, -1), g2.reshape(1, -1), be2.reshape(1, -1),
      W3, b3.reshape(1, -1))


@jax.jit
def _run(x, batch_idx, W1, b1, g1, be1, W2, b2, g2, be2, W3, b3):
    xx = _pool(x, batch_idx)
    return _mlp(xx, W1, b1, g1, be1, W2, b2, g2, be2, W3, b3)


def kernel(x, batch_idx, W1, b1, g1, be1, W2, b2, g2, be2, W3, b3):
    return _run(x, batch_idx, W1, b1, g1, be1, W2, b2, g2, be2, W3, b3)
